# baseline (speedup 1.0000x reference)
import functools

import jax
import jax.numpy as jnp
from jax import lax
from jax.experimental import pallas as pl
from jax.experimental.pallas import tpu as pltpu
from jax.experimental.pallas import tpu_sc as plsc

N_TOK = 32 * 576
DIM = 64
K = 1024
BLK = 512
N_BLKS = N_TOK // BLK
LOSS_SCALE = 1.25 / (N_TOK * DIM)

NW = 32
B_PER_W = N_TOK // NW
CHUNK = 96
N_CHUNKS = B_PER_W // CHUNK


def _vq_idx_body(x_ref, tt_ref, idx_ref, loss_ref, bias_ref):
    i = pl.program_id(0)
    tt = tt_ref[...]

    @pl.when(i == 0)
    def _():
        bias_ref[...] = -0.5 * jnp.sum(tt * tt, axis=0, keepdims=True)
        loss_ref[0, 0] = 0.0

    xb = x_ref[...]
    dots = jax.lax.dot_general(
        xb, tt, (((1,), (0,)), ((), ())),
        preferred_element_type=jnp.float32)
    score = dots + bias_ref[...]
    max_val = jnp.max(score, axis=1, keepdims=True)
    kio = jax.lax.broadcasted_iota(jnp.int32, (BLK, K), 1).astype(jnp.float32)
    first = jnp.where(score == max_val, kio, jnp.float32(K))
    idx_f = jnp.min(first, axis=1, keepdims=True)
    idx_ref[...] = idx_f.astype(jnp.int32)

    x_sq = jnp.sum(xb * xb)
    loss_ref[0, 0] += (x_sq - 2.0 * jnp.sum(max_val)) * LOSS_SCALE


_SC_MESH = plsc.VectorSubcoreMesh(core_axis_name="c", subcore_axis_name="s")


@functools.partial(
    pl.kernel,
    mesh=_SC_MESH,
    out_type=jax.ShapeDtypeStruct((N_TOK, 2 * DIM), jnp.float32),
    scratch_types=(
        [pltpu.VMEM((CHUNK,), jnp.int32) for _ in range(N_CHUNKS)]
        + [pltpu.VMEM((B_PER_W, 2 * DIM), jnp.float32), pltpu.SemaphoreType.DMA]
    ),
)
def _sc_gather(table_hbm, idx_hbm, out_hbm, *scratch):
    idx_vs, rows_v, sem = scratch[:N_CHUNKS], scratch[N_CHUNKS], scratch[-1]
    wid = lax.axis_index("s") * 2 + lax.axis_index("c")
    base = wid * B_PER_W
    for j in range(N_CHUNKS):
        pltpu.sync_copy(idx_hbm.at[pl.ds(base + j * CHUNK, CHUNK)], idx_vs[j])
    copies = [
        pltpu.make_async_copy(
            table_hbm.at[idx_vs[j]],
            rows_v.at[pl.ds(j * CHUNK, CHUNK)],
            sem,
        )
        for j in range(N_CHUNKS)
    ]
    for c in copies:
        c.start()
    for c in copies:
        c.wait()
    pltpu.sync_copy(rows_v, out_hbm.at[pl.ds(base, B_PER_W)])


@jax.jit
def kernel(x, table):
    flat_x = x.reshape(N_TOK, DIM)
    tt = table.T
    idx, loss = pl.pallas_call(
        _vq_idx_body,
        grid=(N_BLKS,),
        in_specs=[
            pl.BlockSpec((BLK, DIM), lambda i: (i, 0)),
            pl.BlockSpec((DIM, K), lambda i: (0, 0)),
        ],
        out_specs=[
            pl.BlockSpec((BLK, 1), lambda i: (i, 0)),
            pl.BlockSpec(memory_space=pltpu.SMEM),
        ],
        out_shape=[
            jax.ShapeDtypeStruct((N_TOK, 1), jnp.int32),
            jax.ShapeDtypeStruct((1, 1), jnp.float32),
        ],
        scratch_shapes=[pltpu.VMEM((1, K), jnp.float32)],
    )(flat_x, tt)
    tab_pad = jnp.pad(table, ((0, 0), (0, DIM)))
    q = _sc_gather(tab_pad, idx.reshape(N_TOK))[:, :DIM]
    return q.reshape(x.shape), loss[0, 0], idx


# speedup vs baseline: 1.5212x; 1.5212x over previous
import jax
import jax.numpy as jnp
from jax.experimental import pallas as pl
from jax.experimental.pallas import tpu as pltpu

N_TOK = 32 * 576
DIM = 64
K = 1024
BLK = 1024
N_BLKS = N_TOK // BLK
LOSS_SCALE = 1.25 / (N_TOK * DIM)


def _vq_body(x_ref, tt_ref, tab_ref, out_ref, idx_ref, loss_ref, bias_ref):
    i = pl.program_id(0)
    tt = tt_ref[...]

    @pl.when(i == 0)
    def _():
        bias_ref[...] = -0.5 * jnp.sum(tt * tt, axis=0, keepdims=True)
        loss_ref[0, 0] = 0.0

    xb = x_ref[...]
    dots = jax.lax.dot_general(
        xb, tt, (((1,), (0,)), ((), ())),
        preferred_element_type=jnp.float32)
    score = dots + bias_ref[...]
    max_val = jnp.max(score, axis=1, keepdims=True)
    kio = jax.lax.broadcasted_iota(jnp.int32, (BLK, K), 1).astype(jnp.float32)
    first = jnp.where(score == max_val, kio, jnp.float32(K))
    idx_f = jnp.min(first, axis=1, keepdims=True)
    idx_ref[...] = idx_f.astype(jnp.int32)
    oh = jnp.where(kio == idx_f, 1.0, 0.0)
    out_ref[...] = jax.lax.dot_general(
        oh, tab_ref[...], (((1,), (0,)), ((), ())),
        preferred_element_type=jnp.float32)

    x_sq = jnp.sum(xb * xb)
    loss_ref[0, 0] += (x_sq - 2.0 * jnp.sum(max_val)) * LOSS_SCALE


@jax.jit
def kernel(x, table):
    flat_x = x.reshape(N_TOK, DIM)
    tt = table.T
    out, idx, loss = pl.pallas_call(
        _vq_body,
        grid=(N_BLKS,),
        in_specs=[
            pl.BlockSpec((BLK, DIM), lambda i: (i, 0)),
            pl.BlockSpec((DIM, K), lambda i: (0, 0)),
            pl.BlockSpec((K, DIM), lambda i: (0, 0)),
        ],
        out_specs=[
            pl.BlockSpec((BLK, DIM), lambda i: (i, 0)),
            pl.BlockSpec((BLK, 1), lambda i: (i, 0)),
            pl.BlockSpec(memory_space=pltpu.SMEM),
        ],
        out_shape=[
            jax.ShapeDtypeStruct((N_TOK, DIM), jnp.float32),
            jax.ShapeDtypeStruct((N_TOK, 1), jnp.int32),
            jax.ShapeDtypeStruct((1, 1), jnp.float32),
        ],
        scratch_shapes=[pltpu.VMEM((1, K), jnp.float32)],
    )(flat_x, tt, table)
    return out.reshape(x.shape), loss[0, 0], idx


# speedup vs baseline: 1.6143x; 1.0612x over previous
import jax
import jax.numpy as jnp
from jax.experimental import pallas as pl
from jax.experimental.pallas import tpu as pltpu

N_TOK = 32 * 576
DIM = 64
K = 1024
BLK = 2304
N_BLKS = N_TOK // BLK
LOSS_SCALE = 1.25 / (N_TOK * DIM)


def _vq_body(x_ref, tt_ref, tab_ref, out_ref, idx_ref, loss_ref, bias_ref):
    i = pl.program_id(0)
    tt = tt_ref[...]

    @pl.when(i == 0)
    def _():
        bias_ref[...] = -0.5 * jnp.sum(tt * tt, axis=0, keepdims=True)
        loss_ref[0, 0] = 0.0

    xb = x_ref[...]
    dots = jax.lax.dot_general(
        xb, tt, (((1,), (0,)), ((), ())),
        preferred_element_type=jnp.float32)
    score = dots + bias_ref[...]
    max_val = jnp.max(score, axis=1, keepdims=True)
    kio = jax.lax.broadcasted_iota(jnp.int32, (BLK, K), 1).astype(jnp.float32)
    first = jnp.where(score == max_val, kio, jnp.float32(K))
    idx_f = jnp.min(first, axis=1, keepdims=True)
    idx_ref[...] = idx_f.astype(jnp.int32)
    oh = jnp.where(kio == idx_f, 1.0, 0.0)
    out_ref[...] = jax.lax.dot_general(
        oh, tab_ref[...], (((1,), (0,)), ((), ())),
        preferred_element_type=jnp.float32)

    x_sq = jnp.sum(xb * xb)
    loss_ref[0, 0] += (x_sq - 2.0 * jnp.sum(max_val)) * LOSS_SCALE


@jax.jit
def kernel(x, table):
    flat_x = x.reshape(N_TOK, DIM)
    tt = table.T
    out, idx, loss = pl.pallas_call(
        _vq_body,
        grid=(N_BLKS,),
        in_specs=[
            pl.BlockSpec((BLK, DIM), lambda i: (i, 0)),
            pl.BlockSpec((DIM, K), lambda i: (0, 0)),
            pl.BlockSpec((K, DIM), lambda i: (0, 0)),
        ],
        out_specs=[
            pl.BlockSpec((BLK, DIM), lambda i: (i, 0)),
            pl.BlockSpec((BLK, 1), lambda i: (i, 0)),
            pl.BlockSpec(memory_space=pltpu.SMEM),
        ],
        out_shape=[
            jax.ShapeDtypeStruct((N_TOK, DIM), jnp.float32),
            jax.ShapeDtypeStruct((N_TOK, 1), jnp.int32),
            jax.ShapeDtypeStruct((1, 1), jnp.float32),
        ],
        scratch_shapes=[pltpu.VMEM((1, K), jnp.float32)],
    )(flat_x, tt, table)
    return out.reshape(x.shape), loss[0, 0], idx


# speedup vs baseline: 1.6485x; 1.0211x over previous
import jax
import jax.numpy as jnp
from jax.experimental import pallas as pl
from jax.experimental.pallas import tpu as pltpu

N_TOK = 32 * 576
DIM = 64
K = 1024
BLK = 4608
N_BLKS = N_TOK // BLK
LOSS_SCALE = 1.25 / (N_TOK * DIM)


def _vq_body(x_ref, tt_ref, tab_ref, out_ref, idx_ref, loss_ref, bias_ref):
    i = pl.program_id(0)
    tt = tt_ref[...]

    @pl.when(i == 0)
    def _():
        bias_ref[...] = -0.5 * jnp.sum(tt * tt, axis=0, keepdims=True)
        loss_ref[0, 0] = 0.0

    xb = x_ref[...]
    dots = jax.lax.dot_general(
        xb, tt, (((1,), (0,)), ((), ())),
        preferred_element_type=jnp.float32)
    score = dots + bias_ref[...]
    max_val = jnp.max(score, axis=1, keepdims=True)
    kio = jax.lax.broadcasted_iota(jnp.int32, (BLK, K), 1).astype(jnp.float32)
    first = jnp.where(score == max_val, kio, jnp.float32(K))
    idx_f = jnp.min(first, axis=1, keepdims=True)
    idx_ref[...] = idx_f.astype(jnp.int32)
    oh = jnp.where(kio == idx_f, 1.0, 0.0)
    out_ref[...] = jax.lax.dot_general(
        oh, tab_ref[...], (((1,), (0,)), ((), ())),
        preferred_element_type=jnp.float32)

    x_sq = jnp.sum(xb * xb)
    loss_ref[0, 0] += (x_sq - 2.0 * jnp.sum(max_val)) * LOSS_SCALE


@jax.jit
def kernel(x, table):
    flat_x = x.reshape(N_TOK, DIM)
    tt = table.T
    out, idx, loss = pl.pallas_call(
        _vq_body,
        grid=(N_BLKS,),
        in_specs=[
            pl.BlockSpec((BLK, DIM), lambda i: (i, 0)),
            pl.BlockSpec((DIM, K), lambda i: (0, 0)),
            pl.BlockSpec((K, DIM), lambda i: (0, 0)),
        ],
        out_specs=[
            pl.BlockSpec((BLK, DIM), lambda i: (i, 0)),
            pl.BlockSpec((BLK, 1), lambda i: (i, 0)),
            pl.BlockSpec(memory_space=pltpu.SMEM),
        ],
        out_shape=[
            jax.ShapeDtypeStruct((N_TOK, DIM), jnp.float32),
            jax.ShapeDtypeStruct((N_TOK, 1), jnp.int32),
            jax.ShapeDtypeStruct((1, 1), jnp.float32),
        ],
        scratch_shapes=[pltpu.VMEM((1, K), jnp.float32)],
    )(flat_x, tt, table)
    return out.reshape(x.shape), loss[0, 0], idx
